# Initial kernel scaffold; baseline (speedup 1.0000x reference)
import jax
import jax.numpy as jnp
from jax.experimental import pallas as pl


def kernel(x, edge_index_ss, edge_index_doc_s, rel_weight, loop_weight, h_bias):
    raise NotImplementedError("write your pallas kernel here")


# speedup vs baseline: 3.1591x; 3.1591x over previous
import functools

import jax
import jax.numpy as jnp
from jax import lax
from jax.experimental import pallas as pl
from jax.experimental.pallas import tpu as pltpu
from jax.experimental.pallas import tpu_sc as plsc

N = 10000
D = 128
E = 320000
L = 3
R = 2

NTILE = 16
CHUNK = 128
GB = 16
NG = 10
CPT = NG * GB
EPT = CPT * CHUNK
EP = NTILE * EPT
NP = 10240
STRIPE = NP // NTILE
SC_OUT = STRIPE // CHUNK


def _sc_body(with_deg, h_hbm, src_hbm, dst_hbm, *refs):
    if with_deg:
        (m_hbm, deg_hbm, src_v, dst_v, rows_v, ones_v,
         acc_sh, deg_sh, gsem) = refs
    else:
        m_hbm, src_v, dst_v, rows_v, acc_sh, gsem = refs
    c = lax.axis_index("c")
    s = lax.axis_index("s")

    zeros16 = jnp.zeros((16,), jnp.float32)

    def _zrow(i, _):
        for k in range(D // 16):
            rows_v[i, pl.ds(k * 16, 16)] = zeros16
        return 0

    lax.fori_loop(0, CHUNK, _zrow, 0)
    if with_deg:
        for k in range(D // 16):
            ones_v[pl.ds(k * 16, 16)] = jnp.full((16,), 1.0, jnp.float32)
    for k in range(SC_OUT):
        pltpu.sync_copy(rows_v, acc_sh.at[pl.ds(s * STRIPE + k * CHUNK, CHUNK)])
        if with_deg:
            pltpu.sync_copy(rows_v.at[0],
                            deg_sh.at[pl.ds(s * STRIPE + k * CHUNK, CHUNK)])
    plsc.subcore_barrier()

    def _group(g, _):
        pltpu.sync_copy(src_hbm.at[c, s, pl.ds(g * GB, GB)], src_v)
        pltpu.sync_copy(dst_hbm.at[c, s, pl.ds(g * GB, GB)], dst_v)

        def _step(j, _):
            pltpu.async_copy(h_hbm.at[src_v.at[j]], rows_v, gsem).wait()
            pltpu.sync_copy(rows_v, acc_sh.at[dst_v.at[j]], add=True)
            if with_deg:
                pltpu.sync_copy(ones_v, deg_sh.at[dst_v.at[j]], add=True)
            return 0

        lax.fori_loop(0, GB, _step, 0)
        return 0

    lax.fori_loop(0, NG, _group, 0)
    plsc.subcore_barrier()

    pltpu.sync_copy(acc_sh.at[pl.ds(s * STRIPE, STRIPE)],
                    m_hbm.at[c, pl.ds(s * STRIPE, STRIPE)])
    if with_deg:
        pltpu.sync_copy(deg_sh.at[pl.ds(s * STRIPE, STRIPE)],
                        deg_hbm.at[c, pl.ds(s * STRIPE, STRIPE)])


def _make_sc_agg(with_deg):
    out_type = [jax.ShapeDtypeStruct((R, NP, D), jnp.float32)]
    if with_deg:
        out_type.append(jax.ShapeDtypeStruct((R, NP), jnp.float32))
    scratch = [
        pltpu.VMEM((GB, CHUNK), jnp.int32),
        pltpu.VMEM((GB, CHUNK), jnp.int32),
        pltpu.VMEM((CHUNK, D), jnp.float32),
    ]
    if with_deg:
        scratch.append(pltpu.VMEM((CHUNK,), jnp.float32))
    scratch.append(pltpu.VMEM_SHARED((NP, D), jnp.float32))
    if with_deg:
        scratch.append(pltpu.VMEM_SHARED((NP,), jnp.float32))
    scratch.append(pltpu.SemaphoreType.DMA)
    return pl.kernel(
        functools.partial(_sc_body, with_deg),
        out_type=out_type if with_deg else out_type[0],
        mesh=plsc.VectorSubcoreMesh(core_axis_name="c", subcore_axis_name="s",
                                    num_cores=R, num_subcores=NTILE),
        scratch_types=scratch,
    )


def _tc_body(h_ref, m1_ref, m2_ref, i1_ref, i2_ref,
             w1_ref, w2_ref, wl_ref, b_ref, o_ref):
    acc = jnp.dot(m1_ref[...] * i1_ref[...], w1_ref[...],
                  preferred_element_type=jnp.float32)
    acc = acc + jnp.dot(m2_ref[...] * i2_ref[...], w2_ref[...],
                        preferred_element_type=jnp.float32)
    acc = acc + jnp.dot(h_ref[...], wl_ref[...],
                        preferred_element_type=jnp.float32)
    o_ref[...] = jnp.maximum(acc + b_ref[...], 0.0)


_BR = 2048


def _tc_fuse(h, m1, m2, inv1, inv2, w1, w2, wl, b):
    row = pl.BlockSpec((_BR, D), lambda i: (i, 0))
    mat = pl.BlockSpec((D, D), lambda i: (0, 0))
    return pl.pallas_call(
        _tc_body,
        grid=(NP // _BR,),
        in_specs=[row, row, row, row, row, mat, mat, mat,
                  pl.BlockSpec((1, D), lambda i: (0, 0))],
        out_specs=row,
        out_shape=jax.ShapeDtypeStruct((NP, D), jnp.float32),
    )(h, m1, m2, inv1, inv2, w1, w2, wl, b)


def _pack_edges(edge_index):
    src = jnp.concatenate(
        [edge_index[0], jnp.zeros((EP - E,), jnp.int32)]).reshape(NTILE, CPT, CHUNK)
    dst = jnp.concatenate(
        [edge_index[1], jnp.full((EP - E,), N, jnp.int32)]).reshape(NTILE, CPT, CHUNK)
    return src, dst


def kernel(x, edge_index_ss, edge_index_doc_s, rel_weight, loop_weight, h_bias):
    src_ss, dst_ss = _pack_edges(edge_index_ss)
    src_ds, dst_ds = _pack_edges(edge_index_doc_s)
    src_all = jnp.stack([src_ss, src_ds])
    dst_all = jnp.stack([dst_ss, dst_ds])

    h = jnp.concatenate([x, jnp.zeros((NP - N, D), jnp.float32)])

    sc_agg_deg = _make_sc_agg(True)
    sc_agg = _make_sc_agg(False)

    m, deg = sc_agg_deg(h, src_all, dst_all)
    inv = 1.0 / jnp.maximum(deg, 1.0)
    inv_bc = jnp.broadcast_to(inv[:, :, None], (R, NP, D))

    for l in range(L):
        if l > 0:
            m = sc_agg(h, src_all, dst_all)
        h = _tc_fuse(h, m[0], m[1], inv_bc[0], inv_bc[1],
                     rel_weight[l, 0], rel_weight[l, 1],
                     loop_weight[l], h_bias[l][None, :])
    return h[:N]


# speedup vs baseline: 3.3682x; 1.0662x over previous
import functools

import jax
import jax.numpy as jnp
from jax import lax
from jax.experimental import pallas as pl
from jax.experimental.pallas import tpu as pltpu
from jax.experimental.pallas import tpu_sc as plsc

N = 10000
D = 128
E = 320000
L = 3
R = 2

NTILE = 16
CHUNK = 128
NB = 2
GB = 32
NG = 5
CPT = NG * GB
EPT = CPT * CHUNK
EP = NTILE * EPT
NP = 10240
STRIPE = NP // NTILE
SC_OUT = STRIPE // CHUNK


def _sc_body(with_deg, h_hbm, src_hbm, dst_hbm, *refs):
    if with_deg:
        (m_hbm, deg_hbm, src_v, dst_v, rows_v, ones_v,
         acc_sh, deg_sh, gsem, ssem) = refs
    else:
        m_hbm, src_v, dst_v, rows_v, acc_sh, gsem, ssem = refs
    c = lax.axis_index("c")
    s = lax.axis_index("s")

    zeros16 = jnp.zeros((16,), jnp.float32)

    def _zrow(i, _):
        for k in range(D // 16):
            rows_v[0, i, pl.ds(k * 16, 16)] = zeros16
        return 0

    lax.fori_loop(0, CHUNK, _zrow, 0)
    if with_deg:
        for k in range(D // 16):
            ones_v[pl.ds(k * 16, 16)] = jnp.full((16,), 1.0, jnp.float32)
    for k in range(SC_OUT):
        pltpu.sync_copy(rows_v.at[0],
                        acc_sh.at[pl.ds(s * STRIPE + k * CHUNK, CHUNK)])
        if with_deg:
            pltpu.sync_copy(rows_v.at[0, 0],
                            deg_sh.at[pl.ds(s * STRIPE + k * CHUNK, CHUNK)])
    plsc.subcore_barrier()

    def _group(g, _):
        pltpu.sync_copy(src_hbm.at[c, s, pl.ds(g * GB, GB)], src_v)
        pltpu.sync_copy(dst_hbm.at[c, s, pl.ds(g * GB, GB)], dst_v)

        gd, sd = {}, {}
        waited = set()
        for j in range(NB - 1):
            gd[j] = pltpu.async_copy(h_hbm.at[src_v.at[j]], rows_v.at[j % NB],
                                     gsem.at[j % NB])
        for j in range(GB):
            b = j % NB
            gd[j].wait()
            sd[j] = pltpu.async_copy(rows_v.at[b], acc_sh.at[dst_v.at[j]],
                                     ssem.at[b], add=True)
            if with_deg:
                pltpu.sync_copy(ones_v, deg_sh.at[dst_v.at[j]], add=True)
            nj = j + NB - 1
            if nj < GB:
                if j >= 1:
                    sd[j - 1].wait()
                    waited.add(j - 1)
                gd[nj] = pltpu.async_copy(h_hbm.at[src_v.at[nj]],
                                          rows_v.at[nj % NB], gsem.at[nj % NB])
        for j in range(GB):
            if j not in waited:
                sd[j].wait()
        return 0

    lax.fori_loop(0, NG, _group, 0)
    plsc.subcore_barrier()

    pltpu.sync_copy(acc_sh.at[pl.ds(s * STRIPE, STRIPE)],
                    m_hbm.at[c, pl.ds(s * STRIPE, STRIPE)])
    if with_deg:
        pltpu.sync_copy(deg_sh.at[pl.ds(s * STRIPE, STRIPE)],
                        deg_hbm.at[c, pl.ds(s * STRIPE, STRIPE)])


def _make_sc_agg(with_deg):
    out_type = [jax.ShapeDtypeStruct((R, NP, D), jnp.float32)]
    if with_deg:
        out_type.append(jax.ShapeDtypeStruct((R, NP), jnp.float32))
    scratch = [
        pltpu.VMEM((GB, CHUNK), jnp.int32),
        pltpu.VMEM((GB, CHUNK), jnp.int32),
        pltpu.VMEM((NB, CHUNK, D), jnp.float32),
    ]
    if with_deg:
        scratch.append(pltpu.VMEM((CHUNK,), jnp.float32))
    scratch.append(pltpu.VMEM_SHARED((NP, D), jnp.float32))
    if with_deg:
        scratch.append(pltpu.VMEM_SHARED((NP,), jnp.float32))
    scratch.append(pltpu.SemaphoreType.DMA((NB,)))
    scratch.append(pltpu.SemaphoreType.DMA((NB,)))
    return pl.kernel(
        functools.partial(_sc_body, with_deg),
        out_type=out_type if with_deg else out_type[0],
        mesh=plsc.VectorSubcoreMesh(core_axis_name="c", subcore_axis_name="s",
                                    num_cores=R, num_subcores=NTILE),
        scratch_types=scratch,
    )


def _tc_body(h_ref, m1_ref, m2_ref, i1_ref, i2_ref,
             w1_ref, w2_ref, wl_ref, b_ref, o_ref):
    acc = jnp.dot(m1_ref[...] * i1_ref[...], w1_ref[...],
                  preferred_element_type=jnp.float32)
    acc = acc + jnp.dot(m2_ref[...] * i2_ref[...], w2_ref[...],
                        preferred_element_type=jnp.float32)
    acc = acc + jnp.dot(h_ref[...], wl_ref[...],
                        preferred_element_type=jnp.float32)
    o_ref[...] = jnp.maximum(acc + b_ref[...], 0.0)


_BR = 2048


def _tc_fuse(h, m1, m2, inv1, inv2, w1, w2, wl, b):
    row = pl.BlockSpec((_BR, D), lambda i: (i, 0))
    mat = pl.BlockSpec((D, D), lambda i: (0, 0))
    return pl.pallas_call(
        _tc_body,
        grid=(NP // _BR,),
        in_specs=[row, row, row, row, row, mat, mat, mat,
                  pl.BlockSpec((1, D), lambda i: (0, 0))],
        out_specs=row,
        out_shape=jax.ShapeDtypeStruct((NP, D), jnp.float32),
    )(h, m1, m2, inv1, inv2, w1, w2, wl, b)


def _pack_edges(edge_index):
    src = jnp.concatenate(
        [edge_index[0], jnp.zeros((EP - E,), jnp.int32)]).reshape(NTILE, CPT, CHUNK)
    dst = jnp.concatenate(
        [edge_index[1], jnp.full((EP - E,), N, jnp.int32)]).reshape(NTILE, CPT, CHUNK)
    return src, dst


def kernel(x, edge_index_ss, edge_index_doc_s, rel_weight, loop_weight, h_bias):
    src_ss, dst_ss = _pack_edges(edge_index_ss)
    src_ds, dst_ds = _pack_edges(edge_index_doc_s)
    src_all = jnp.stack([src_ss, src_ds])
    dst_all = jnp.stack([dst_ss, dst_ds])

    h = jnp.concatenate([x, jnp.zeros((NP - N, D), jnp.float32)])

    sc_agg_deg = _make_sc_agg(True)
    sc_agg = _make_sc_agg(False)

    m, deg = sc_agg_deg(h, src_all, dst_all)
    inv = 1.0 / jnp.maximum(deg, 1.0)
    inv_bc = jnp.broadcast_to(inv[:, :, None], (R, NP, D))

    for l in range(L):
        if l > 0:
            m = sc_agg(h, src_all, dst_all)
        h = _tc_fuse(h, m[0], m[1], inv_bc[0], inv_bc[1],
                     rel_weight[l, 0], rel_weight[l, 1],
                     loop_weight[l], h_bias[l][None, :])
    return h[:N]


# speedup vs baseline: 3.4735x; 1.0313x over previous
import functools

import jax
import jax.numpy as jnp
from jax import lax
from jax.experimental import pallas as pl
from jax.experimental.pallas import tpu as pltpu
from jax.experimental.pallas import tpu_sc as plsc

N = 10000
D = 128
E = 320000
L = 3
R = 2

NTILE = 16
CHUNK = 64
NB = 4
GB = 40
NG = 8
CPT = NG * GB
EPT = CPT * CHUNK
EP = NTILE * EPT
NP = 10240
STRIPE = NP // NTILE
SC_OUT = STRIPE // CHUNK


def _sc_body(with_deg, h_hbm, src_hbm, dst_hbm, *refs):
    if with_deg:
        (m_hbm, deg_hbm, src_v, dst_v, rows_v, ones_v,
         acc_sh, deg_sh, gsem, ssem) = refs
    else:
        m_hbm, src_v, dst_v, rows_v, acc_sh, gsem, ssem = refs
    c = lax.axis_index("c")
    s = lax.axis_index("s")

    zeros16 = jnp.zeros((16,), jnp.float32)

    def _zrow(i, _):
        for k in range(D // 16):
            rows_v[0, i, pl.ds(k * 16, 16)] = zeros16
        return 0

    lax.fori_loop(0, CHUNK, _zrow, 0)
    if with_deg:
        for k in range(CHUNK // 16):
            ones_v[pl.ds(k * 16, 16)] = jnp.full((16,), 1.0, jnp.float32)
    for k in range(SC_OUT):
        pltpu.sync_copy(rows_v.at[0],
                        acc_sh.at[pl.ds(s * STRIPE + k * CHUNK, CHUNK)])
        if with_deg:
            pltpu.sync_copy(rows_v.at[0, 0, pl.ds(0, CHUNK)],
                            deg_sh.at[pl.ds(s * STRIPE + k * CHUNK, CHUNK)])
    plsc.subcore_barrier()

    def _group(g, _):
        pltpu.sync_copy(src_hbm.at[c, s, pl.ds(g * GB, GB)], src_v)
        pltpu.sync_copy(dst_hbm.at[c, s, pl.ds(g * GB, GB)], dst_v)

        gd, sd = {}, {}
        waited = set()
        for j in range(NB - 1):
            gd[j] = pltpu.async_copy(h_hbm.at[src_v.at[j]], rows_v.at[j % NB],
                                     gsem.at[j % NB])
        for j in range(GB):
            b = j % NB
            gd[j].wait()
            sd[j] = pltpu.async_copy(rows_v.at[b], acc_sh.at[dst_v.at[j]],
                                     ssem.at[b], add=True)
            if with_deg:
                pltpu.sync_copy(ones_v, deg_sh.at[dst_v.at[j]], add=True)
            nj = j + NB - 1
            if nj < GB:
                if j >= 1:
                    sd[j - 1].wait()
                    waited.add(j - 1)
                gd[nj] = pltpu.async_copy(h_hbm.at[src_v.at[nj]],
                                          rows_v.at[nj % NB], gsem.at[nj % NB])
        for j in range(GB):
            if j not in waited:
                sd[j].wait()
        return 0

    lax.fori_loop(0, NG, _group, 0)
    plsc.subcore_barrier()

    pltpu.sync_copy(acc_sh.at[pl.ds(s * STRIPE, STRIPE)],
                    m_hbm.at[c, pl.ds(s * STRIPE, STRIPE)])
    if with_deg:
        pltpu.sync_copy(deg_sh.at[pl.ds(s * STRIPE, STRIPE)],
                        deg_hbm.at[c, pl.ds(s * STRIPE, STRIPE)])


def _make_sc_agg(with_deg):
    out_type = [jax.ShapeDtypeStruct((R, NP, D), jnp.float32)]
    if with_deg:
        out_type.append(jax.ShapeDtypeStruct((R, NP), jnp.float32))
    scratch = [
        pltpu.VMEM((GB, CHUNK), jnp.int32),
        pltpu.VMEM((GB, CHUNK), jnp.int32),
        pltpu.VMEM((NB, CHUNK, D), jnp.float32),
    ]
    if with_deg:
        scratch.append(pltpu.VMEM((CHUNK,), jnp.float32))
    scratch.append(pltpu.VMEM_SHARED((NP, D), jnp.float32))
    if with_deg:
        scratch.append(pltpu.VMEM_SHARED((NP,), jnp.float32))
    scratch.append(pltpu.SemaphoreType.DMA((NB,)))
    scratch.append(pltpu.SemaphoreType.DMA((NB,)))
    return pl.kernel(
        functools.partial(_sc_body, with_deg),
        out_type=out_type if with_deg else out_type[0],
        mesh=plsc.VectorSubcoreMesh(core_axis_name="c", subcore_axis_name="s",
                                    num_cores=R, num_subcores=NTILE),
        scratch_types=scratch,
    )


def _tc_body(h_ref, m1_ref, m2_ref, i1_ref, i2_ref,
             w1_ref, w2_ref, wl_ref, b_ref, o_ref):
    acc = jnp.dot(m1_ref[...] * i1_ref[...], w1_ref[...],
                  preferred_element_type=jnp.float32)
    acc = acc + jnp.dot(m2_ref[...] * i2_ref[...], w2_ref[...],
                        preferred_element_type=jnp.float32)
    acc = acc + jnp.dot(h_ref[...], wl_ref[...],
                        preferred_element_type=jnp.float32)
    o_ref[...] = jnp.maximum(acc + b_ref[...], 0.0)


_BR = 2048


def _tc_fuse(h, m1, m2, inv1, inv2, w1, w2, wl, b):
    row = pl.BlockSpec((_BR, D), lambda i: (i, 0))
    mat = pl.BlockSpec((D, D), lambda i: (0, 0))
    return pl.pallas_call(
        _tc_body,
        grid=(NP // _BR,),
        in_specs=[row, row, row, row, row, mat, mat, mat,
                  pl.BlockSpec((1, D), lambda i: (0, 0))],
        out_specs=row,
        out_shape=jax.ShapeDtypeStruct((NP, D), jnp.float32),
    )(h, m1, m2, inv1, inv2, w1, w2, wl, b)


def _pack_edges(edge_index):
    src = jnp.concatenate(
        [edge_index[0], jnp.zeros((EP - E,), jnp.int32)]).reshape(NTILE, CPT, CHUNK)
    dst = jnp.concatenate(
        [edge_index[1], jnp.full((EP - E,), N, jnp.int32)]).reshape(NTILE, CPT, CHUNK)
    return src, dst


def kernel(x, edge_index_ss, edge_index_doc_s, rel_weight, loop_weight, h_bias):
    src_ss, dst_ss = _pack_edges(edge_index_ss)
    src_ds, dst_ds = _pack_edges(edge_index_doc_s)
    src_all = jnp.stack([src_ss, src_ds])
    dst_all = jnp.stack([dst_ss, dst_ds])

    h = jnp.concatenate([x, jnp.zeros((NP - N, D), jnp.float32)])

    sc_agg_deg = _make_sc_agg(True)
    sc_agg = _make_sc_agg(False)

    m, deg = sc_agg_deg(h, src_all, dst_all)
    inv = 1.0 / jnp.maximum(deg, 1.0)
    inv_bc = jnp.broadcast_to(inv[:, :, None], (R, NP, D))

    for l in range(L):
        if l > 0:
            m = sc_agg(h, src_all, dst_all)
        h = _tc_fuse(h, m[0], m[1], inv_bc[0], inv_bc[1],
                     rel_weight[l, 0], rel_weight[l, 1],
                     loop_weight[l], h_bias[l][None, :])
    return h[:N]
